# baseline (speedup 1.0000x reference)
import functools

import jax
import jax.numpy as jnp
from jax import lax
from jax.experimental import pallas as pl
from jax.experimental.pallas import tpu as pltpu
from jax.experimental.pallas import tpu_sc as plsc

NF = 10
NV = 500
H = 128
NC = 2
NS = 16
NW = NC * NS
C = 128


def _sc_lookup_sum(n, nch, slots):
    mesh = plsc.VectorSubcoreMesh(core_axis_name="c", subcore_axis_name="s")
    rem_workers = nch - NW * (slots - 1)

    @functools.partial(
        pl.kernel,
        out_type=jax.ShapeDtypeStruct((n, H), jnp.bfloat16),
        mesh=mesh,
        scratch_types=[
            pltpu.VMEM((2, NF, 1, C), jnp.int32),
            pltpu.VMEM((2, C, H), jnp.bfloat16),
            pltpu.VMEM_SHARED((NF * NV, H), jnp.bfloat16),
            pltpu.SemaphoreType.DMA,
            pltpu.SemaphoreType.DMA,
            pltpu.SemaphoreType.DMA,
        ],
        compiler_params=pltpu.CompilerParams(use_tc_tiling_on_sc=False),
    )
    def body(x_hbm, tab_hbm, out_hbm, idx_v, acc_v, sh_tab,
             sem_g, sem_i, sem_o):
        sid = lax.axis_index("s")
        wid = sid * NC + lax.axis_index("c")

        @pl.when(sid == 0)
        def _stage():
            pltpu.sync_copy(tab_hbm, sh_tab)
        plsc.subcore_barrier()

        pltpu.async_copy(x_hbm.at[0, wid], idx_v.at[0], sem_i)

        def chunk_body(j, carry):
            p = lax.rem(j, 2)
            idx = idx_v.at[p]
            acc = acc_v.at[p]
            g = j * NW + wid
            base = jnp.minimum(g * C, n - C)

            pltpu.make_async_copy(x_hbm.at[j, wid], idx, sem_i).wait()

            @pl.when(j + 1 < slots)
            def _prefetch():
                pltpu.async_copy(x_hbm.at[j + 1, wid], idx_v.at[1 - p], sem_i)

            @pl.when(j >= 2)
            def _drain():
                pltpu.make_async_copy(
                    acc, out_hbm.at[pl.ds(0, C)], sem_o).wait()

            @pl.when(g < nch)
            def _work():
                pltpu.async_copy(sh_tab.at[idx.at[0, 0]], acc, sem_g).wait()
                cps = [pltpu.async_copy(sh_tab.at[idx.at[f, 0]], acc, sem_g,
                                        add=True)
                       for f in range(1, NF)]
                for cp in cps:
                    cp.wait()
                pltpu.async_copy(acc, out_hbm.at[pl.ds(base, C)], sem_o)
            return carry

        lax.fori_loop(0, slots, chunk_body, 0)

        pltpu.make_async_copy(acc_v.at[0], out_hbm.at[pl.ds(0, C)],
                              sem_o).wait()

        @pl.when(wid < rem_workers)
        def _last_drain():
            pltpu.make_async_copy(acc_v.at[0], out_hbm.at[pl.ds(0, C)],
                                  sem_o).wait()

    return body


def kernel(x, emb_0, emb_1, emb_2, emb_3, emb_4, emb_5, emb_6, emb_7,
           emb_8, emb_9):
    n = x.shape[0]
    full, rem = divmod(n, C)
    nch = full + (1 if rem else 0)
    slots = -(-nch // NW)
    xi = x.reshape(n, NF)
    if rem:
        xi = jnp.concatenate([xi[:full * C], xi[n - C:]], axis=0)
    if slots * NW > nch:
        xi = jnp.concatenate(
            [xi, jnp.zeros(((slots * NW - nch) * C, NF), jnp.int32)], axis=0)
    xb = xi.reshape(slots, NW, C, NF).transpose(0, 1, 3, 2)
    xb = xb + (jnp.arange(NF, dtype=jnp.int32) * NV).reshape(1, 1, NF, 1)
    xb = xb.reshape(slots, NW, NF, 1, C)

    tab = jnp.concatenate([emb_0, emb_1, emb_2, emb_3, emb_4, emb_5, emb_6,
                           emb_7, emb_8, emb_9], axis=0)
    tab = tab.astype(jnp.bfloat16)

    out = _sc_lookup_sum(n, nch, slots)(xb, tab)
    return out.astype(jnp.float32).reshape(n, 1, H)


# speedup vs baseline: 1.1541x; 1.1541x over previous
import functools

import jax
import jax.numpy as jnp
from jax import lax
from jax.experimental import pallas as pl
from jax.experimental.pallas import tpu as pltpu
from jax.experimental.pallas import tpu_sc as plsc

NF = 10
NV = 500
H = 128
NC = 2
NS = 16
NW = NC * NS
C = 128


def _sc_lookup_sum(n, nch, slots):
    mesh = plsc.VectorSubcoreMesh(core_axis_name="c", subcore_axis_name="s")
    rem_workers = nch - NW * (slots - 1)

    @functools.partial(
        pl.kernel,
        out_type=jax.ShapeDtypeStruct((n, H), jnp.float32),
        mesh=mesh,
        scratch_types=[
            pltpu.VMEM((2, NF, 1, C), jnp.int32),
            pltpu.VMEM((2, C, H), jnp.float32),
            pltpu.VMEM_SHARED((NF * NV, H), jnp.float32),
            pltpu.SemaphoreType.DMA,
            pltpu.SemaphoreType.DMA,
            pltpu.SemaphoreType.DMA,
            pltpu.SemaphoreType.DMA,
        ],
    )
    def body(x_hbm, tab_hbm, out_hbm, idx_v, acc_v, sh_tab,
             sem_f, sem_g, sem_i, sem_o):
        sid = lax.axis_index("s")
        wid = sid * NC + lax.axis_index("c")

        @pl.when(sid == 0)
        def _stage():
            pltpu.sync_copy(tab_hbm, sh_tab)
        plsc.subcore_barrier()

        pltpu.async_copy(x_hbm.at[0, wid], idx_v.at[0], sem_i)

        def add_cps(idx, acc):
            return [pltpu.async_copy(sh_tab.at[idx.at[f, 0]], acc, sem_g,
                                     add=True)
                    for f in range(1, NF)]

        def chunk_body(j, carry):
            p = lax.rem(j, 2)
            q = 1 - p
            idx = idx_v.at[p]
            acc = acc_v.at[p]
            g = j * NW + wid
            base = jnp.minimum(g * C, n - C)
            prev_base = jnp.minimum((g - NW) * C, n - C)

            pltpu.make_async_copy(x_hbm.at[j, wid], idx, sem_i).wait()

            @pl.when(j >= 2)
            def _drain():
                pltpu.make_async_copy(
                    acc, out_hbm.at[pl.ds(0, C)], sem_o).wait()

            @pl.when(g < nch)
            def _fire0():
                pltpu.async_copy(sh_tab.at[idx.at[0, 0]], acc, sem_f)

            @pl.when(j > 0)
            def _retire_prev():
                for _ in range(NF - 1):
                    pltpu.make_async_copy(out_hbm.at[pl.ds(0, C)],
                                          acc_v.at[q], sem_g).wait()
                pltpu.async_copy(
                    acc_v.at[q], out_hbm.at[pl.ds(prev_base, C)], sem_o)

            @pl.when(j + 1 < slots)
            def _prefetch():
                pltpu.async_copy(x_hbm.at[j + 1, wid], idx_v.at[q], sem_i)

            @pl.when(g < nch)
            def _fire_adds():
                pltpu.make_async_copy(
                    sh_tab.at[idx.at[0, 0]], acc, sem_f).wait()
                add_cps(idx, acc)
            return carry

        lax.fori_loop(0, slots, chunk_body, 0)

        p_last = lax.rem(slots - 1, 2)
        g_last = (slots - 1) * NW + wid

        @pl.when(g_last < nch)
        def _retire_last():
            for _ in range(NF - 1):
                pltpu.make_async_copy(out_hbm.at[pl.ds(0, C)],
                                      acc_v.at[p_last], sem_g).wait()
            pltpu.async_copy(
                acc_v.at[p_last],
                out_hbm.at[pl.ds(jnp.minimum(g_last * C, n - C), C)], sem_o)

        pltpu.make_async_copy(acc_v.at[0], out_hbm.at[pl.ds(0, C)],
                              sem_o).wait()

        @pl.when(wid < rem_workers)
        def _last_drain():
            pltpu.make_async_copy(acc_v.at[0], out_hbm.at[pl.ds(0, C)],
                                  sem_o).wait()

    return body


def kernel(x, emb_0, emb_1, emb_2, emb_3, emb_4, emb_5, emb_6, emb_7,
           emb_8, emb_9):
    n = x.shape[0]
    full, rem = divmod(n, C)
    nch = full + (1 if rem else 0)
    slots = -(-nch // NW)
    xi = x.reshape(n, NF)
    if rem:
        xi = jnp.concatenate([xi[:full * C], xi[n - C:]], axis=0)
    if slots * NW > nch:
        xi = jnp.concatenate(
            [xi, jnp.zeros(((slots * NW - nch) * C, NF), jnp.int32)], axis=0)
    xb = xi.reshape(slots, NW, C, NF).transpose(0, 1, 3, 2)
    xb = xb + (jnp.arange(NF, dtype=jnp.int32) * NV).reshape(1, 1, NF, 1)
    xb = xb.reshape(slots, NW, NF, 1, C)

    tab = jnp.concatenate([emb_0, emb_1, emb_2, emb_3, emb_4, emb_5, emb_6,
                           emb_7, emb_8, emb_9], axis=0)

    out = _sc_lookup_sum(n, nch, slots)(xb, tab)
    return out.reshape(n, 1, H)


# speedup vs baseline: 1.2008x; 1.0405x over previous
import functools

import jax
import jax.numpy as jnp
from jax import lax
from jax.experimental import pallas as pl
from jax.experimental.pallas import tpu as pltpu
from jax.experimental.pallas import tpu_sc as plsc

NF = 10
NV = 500
H = 128
NC = 2
NS = 16
NW = NC * NS
C = 128


def _sc_lookup_sum(n, nch, slots):
    mesh = plsc.VectorSubcoreMesh(core_axis_name="c", subcore_axis_name="s")
    rem_workers = nch - NW * (slots - 1)

    @functools.partial(
        pl.kernel,
        out_type=jax.ShapeDtypeStruct((n, H), jnp.float32),
        mesh=mesh,
        scratch_types=[
            pltpu.VMEM((2, NF, 1, C), jnp.int32),
            pltpu.VMEM((2, C, H), jnp.float32),
            pltpu.VMEM_SHARED((NF * NV, H), jnp.float32),
            pltpu.SemaphoreType.DMA,
            pltpu.SemaphoreType.DMA,
            pltpu.SemaphoreType.DMA,
        ],
    )
    def body(x_hbm, tab_hbm, out_hbm, idx_v, acc_v, sh_tab,
             sem_g, sem_i, sem_o):
        sid = lax.axis_index("s")
        wid = sid * NC + lax.axis_index("c")

        @pl.when(sid == 0)
        def _stage():
            pltpu.sync_copy(tab_hbm, sh_tab)
        plsc.subcore_barrier()

        pltpu.async_copy(x_hbm.at[0, wid], idx_v.at[0], sem_i)

        def chunk_body(j, carry):
            p = lax.rem(j, 2)
            idx = idx_v.at[p]
            acc = acc_v.at[p]
            g = j * NW + wid
            base = jnp.minimum(g * C, n - C)

            pltpu.make_async_copy(x_hbm.at[j, wid], idx, sem_i).wait()

            @pl.when(j + 1 < slots)
            def _prefetch():
                pltpu.async_copy(x_hbm.at[j + 1, wid], idx_v.at[1 - p], sem_i)

            @pl.when(j >= 2)
            def _drain():
                pltpu.make_async_copy(
                    acc, out_hbm.at[pl.ds(0, C)], sem_o).wait()

            @pl.when(g < nch)
            def _work():
                pltpu.async_copy(sh_tab.at[idx.at[0, 0]], acc, sem_g).wait()
                cps = [pltpu.async_copy(sh_tab.at[idx.at[f, 0]], acc, sem_g,
                                        add=True)
                       for f in range(1, NF)]
                for cp in cps:
                    cp.wait()
                pltpu.async_copy(acc, out_hbm.at[pl.ds(base, C)], sem_o)
            return carry

        lax.fori_loop(0, slots, chunk_body, 0)

        pltpu.make_async_copy(acc_v.at[0], out_hbm.at[pl.ds(0, C)],
                              sem_o).wait()

        @pl.when(wid < rem_workers)
        def _last_drain():
            pltpu.make_async_copy(acc_v.at[0], out_hbm.at[pl.ds(0, C)],
                                  sem_o).wait()

    return body


def kernel(x, emb_0, emb_1, emb_2, emb_3, emb_4, emb_5, emb_6, emb_7,
           emb_8, emb_9):
    n = x.shape[0]
    full, rem = divmod(n, C)
    nch = full + (1 if rem else 0)
    slots = -(-nch // NW)
    xi = x.reshape(n, NF)
    if rem:
        xi = jnp.concatenate([xi[:full * C], xi[n - C:]], axis=0)
    if slots * NW > nch:
        xi = jnp.concatenate(
            [xi, jnp.zeros(((slots * NW - nch) * C, NF), jnp.int32)], axis=0)
    xb = xi.reshape(slots, NW, C, NF).transpose(0, 1, 3, 2)
    xb = xb + (jnp.arange(NF, dtype=jnp.int32) * NV).reshape(1, 1, NF, 1)
    xb = xb.reshape(slots, NW, NF, 1, C)

    tab = jnp.concatenate([emb_0, emb_1, emb_2, emb_3, emb_4, emb_5, emb_6,
                           emb_7, emb_8, emb_9], axis=0)

    out = _sc_lookup_sum(n, nch, slots)(xb, tab)
    return out.reshape(n, 1, H)
